# Initial kernel scaffold; baseline (speedup 1.0000x reference)
import jax
import jax.numpy as jnp
from jax.experimental import pallas as pl


def kernel(encoder_output, durations):
    raise NotImplementedError("write your pallas kernel here")


# speedup vs baseline: 17.4457x; 17.4457x over previous
import jax
import jax.numpy as jnp
from jax import lax
from jax.experimental import pallas as pl
from jax.experimental.pallas import tpu as pltpu
from jax.experimental.pallas import tpu_sc as plsc

B, T, D = 16, 512, 256
L = 2048
MAX_DUR = 8

NC, NS = 2, 16
NW = NC * NS
HALF = L // 2
CHUNK = 128
NCHUNK = HALF // CHUNK
LANES = 16


def _body(enc_hbm, dur_hbm, out_hbm, dur_v, idx_v, rows_v, sem):
  cid = lax.axis_index("c")
  sid = lax.axis_index("s")
  wid = sid * NC + cid
  b = wid // 2
  half = wid % 2
  pbase = half * HALF

  pltpu.sync_copy(dur_hbm.at[b], dur_v)

  zi = jnp.zeros((LANES,), jnp.int32)
  for j in range(HALF // LANES):
    idx_v[pl.ds(pbase + j * LANES, LANES)] = zi

  t_iota = lax.iota(jnp.int32, LANES)
  carry = jnp.int32(0)
  row0 = b * T
  for i in range(T // LANES):
    v = dur_v[pl.ds(i * LANES, LANES)]
    inc = plsc.cumsum(v)
    start = inc - v + carry
    carry = carry + jnp.sum(v)
    val = t_iota + (row0 + i * LANES)
    for k in range(MAX_DUR - 1):
      pos = start + k
      mask = (v > k) & (pos < L)
      plsc.store_scatter(idx_v, [pos], val, mask=mask)
  total = carry

  zf = jnp.zeros((LANES,), jnp.float32)
  for ci in range(NCHUNK):
    off = pbase + ci * CHUNK
    pltpu.async_copy(enc_hbm.at[idx_v.at[pl.ds(off, CHUNK)]], rows_v, sem).wait()

    m0 = jnp.clip(total - off, 0, CHUNK)

    def _zero_row(r, acc):
      for j in range(D // LANES):
        rows_v[r, pl.ds(j * LANES, LANES)] = zf
      return acc

    lax.fori_loop(m0, CHUNK, _zero_row, 0)

    pltpu.sync_copy(rows_v, out_hbm.at[pl.ds(b * L + off, CHUNK), :])


@jax.jit
def kernel(encoder_output, durations):
  enc_flat = encoder_output.reshape(B * T, D)
  dur32 = durations.astype(jnp.int32)
  mesh = plsc.VectorSubcoreMesh(
      core_axis_name="c", subcore_axis_name="s", num_cores=NC, num_subcores=NS
  )
  run = pl.kernel(
      _body,
      out_type=jax.ShapeDtypeStruct((B * L, D), jnp.float32),
      mesh=mesh,
      scratch_types=[
          pltpu.VMEM((T,), jnp.int32),
          pltpu.VMEM((L,), jnp.int32),
          pltpu.VMEM((CHUNK, D), jnp.float32),
          pltpu.SemaphoreType.DMA,
      ],
      compiler_params=pltpu.CompilerParams(needs_layout_passes=False),
  )
  out = run(enc_flat, dur32)
  return out.reshape(B, L, D)


# speedup vs baseline: 17.7540x; 1.0177x over previous
import jax
import jax.numpy as jnp
from jax import lax
from jax.experimental import pallas as pl
from jax.experimental.pallas import tpu as pltpu
from jax.experimental.pallas import tpu_sc as plsc

B, T, D = 16, 512, 256
L = 2048
MAX_DUR = 8

NC, NS = 2, 16
NW = NC * NS
HALF = L // 2
CHUNK = 128
NCHUNK = HALF // CHUNK
LANES = 16


def _body(enc_hbm, dur_hbm, out_hbm, dur_v, idx_v, rows_a, rows_b, gsem_a, gsem_b, wsem_a, wsem_b):
  cid = lax.axis_index("c")
  sid = lax.axis_index("s")
  wid = sid * NC + cid
  b = wid // 2
  half = wid % 2
  pbase = half * HALF

  pltpu.sync_copy(dur_hbm.at[b], dur_v)

  zi = jnp.zeros((LANES,), jnp.int32)
  for j in range(HALF // LANES):
    idx_v[pl.ds(pbase + j * LANES, LANES)] = zi

  t_iota = lax.iota(jnp.int32, LANES)
  carry = jnp.int32(0)
  row0 = b * T
  for i in range(T // LANES):
    v = dur_v[pl.ds(i * LANES, LANES)]
    inc = plsc.cumsum(v)
    start = inc - v + carry
    carry = carry + jnp.sum(v)
    val = t_iota + (row0 + i * LANES)
    for k in range(MAX_DUR - 1):
      pos = start + k
      mask = (v > k) & (pos < L)
      plsc.store_scatter(idx_v, [pos], val, mask=mask)
  total = carry

  zf = jnp.zeros((LANES,), jnp.float32)
  bufs = (rows_a, rows_b)
  gsems = (gsem_a, gsem_b)
  wsems = (wsem_a, wsem_b)

  def _gather(ci, buf, sem):
    off = pbase + ci * CHUNK
    return pltpu.async_copy(enc_hbm.at[idx_v.at[pl.ds(off, CHUNK)]], buf, sem)

  g = [None, None]
  w = [None, None]
  g[0] = _gather(0, bufs[0], gsems[0])
  for ci in range(NCHUNK):
    cur = ci % 2
    nxt = (ci + 1) % 2
    if ci + 1 < NCHUNK:
      if w[nxt] is not None:
        w[nxt].wait()
      g[nxt] = _gather(ci + 1, bufs[nxt], gsems[nxt])
    g[cur].wait()

    off = pbase + ci * CHUNK
    m0 = jnp.clip(total - off, 0, CHUNK)
    buf = bufs[cur]

    def _zero_row(r, acc):
      for j in range(D // LANES):
        buf[r, pl.ds(j * LANES, LANES)] = zf
      return acc

    lax.fori_loop(m0, CHUNK, _zero_row, 0)

    w[cur] = pltpu.async_copy(buf, out_hbm.at[pl.ds(b * L + off, CHUNK), :], wsems[cur])
  w[0].wait()
  w[1].wait()


@jax.jit
def kernel(encoder_output, durations):
  enc_flat = encoder_output.reshape(B * T, D)
  dur32 = durations.astype(jnp.int32)
  mesh = plsc.VectorSubcoreMesh(
      core_axis_name="c", subcore_axis_name="s", num_cores=NC, num_subcores=NS
  )
  run = pl.kernel(
      _body,
      out_type=jax.ShapeDtypeStruct((B * L, D), jnp.float32),
      mesh=mesh,
      scratch_types=[
          pltpu.VMEM((T,), jnp.int32),
          pltpu.VMEM((L,), jnp.int32),
          pltpu.VMEM((CHUNK, D), jnp.float32),
          pltpu.VMEM((CHUNK, D), jnp.float32),
          pltpu.SemaphoreType.DMA,
          pltpu.SemaphoreType.DMA,
          pltpu.SemaphoreType.DMA,
          pltpu.SemaphoreType.DMA,
      ],
      compiler_params=pltpu.CompilerParams(needs_layout_passes=False),
  )
  out = run(enc_flat, dur32)
  return out.reshape(B, L, D)


# speedup vs baseline: 17.8364x; 1.0046x over previous
import jax
import jax.numpy as jnp
from jax import lax
from jax.experimental import pallas as pl
from jax.experimental.pallas import tpu as pltpu
from jax.experimental.pallas import tpu_sc as plsc

B, T, D = 16, 512, 256
L = 2048
MAX_DUR = 8

NC, NS = 2, 16
HALF = L // 2
CHUNK = 64
NCHUNK = HALF // CHUNK
NBUF = 4
LANES = 16


def _body(enc_hbm, dur_hbm, out_hbm, dur_v, idx_v, b0, b1, b2, b3,
          g0, g1, g2, g3, w0, w1, w2, w3):
  cid = lax.axis_index("c")
  sid = lax.axis_index("s")
  wid = sid * NC + cid
  b = wid // 2
  half = wid % 2
  pbase = half * HALF

  pltpu.sync_copy(dur_hbm.at[b], dur_v)

  zi = jnp.zeros((LANES,), jnp.int32)
  for j in range(HALF // LANES):
    idx_v[pl.ds(pbase + j * LANES, LANES)] = zi

  t_iota = lax.iota(jnp.int32, LANES)
  carry = jnp.int32(0)
  row0 = b * T
  for i in range(T // LANES):
    v = dur_v[pl.ds(i * LANES, LANES)]
    inc = plsc.cumsum(v)
    start = inc - v + carry
    carry = carry + jnp.sum(v)
    val = t_iota + (row0 + i * LANES)
    for k in range(MAX_DUR - 1):
      pos = start + k
      mask = (v > k) & (pos < L)
      plsc.store_scatter(idx_v, [pos], val, mask=mask)
  total = carry

  zf = jnp.zeros((LANES,), jnp.float32)
  bufs = (b0, b1, b2, b3)
  gsems = (g0, g1, g2, g3)
  wsems = (w0, w1, w2, w3)

  def _gather(ci, buf, sem):
    off = pbase + ci * CHUNK
    return pltpu.async_copy(enc_hbm.at[idx_v.at[pl.ds(off, CHUNK)]], buf, sem)

  g = [None] * NBUF
  w = [None] * NBUF
  for pre in range(NBUF - 1):
    g[pre] = _gather(pre, bufs[pre], gsems[pre])
  for ci in range(NCHUNK):
    s = ci % NBUF
    ni = ci + NBUF - 1
    if ni < NCHUNK:
      ns = ni % NBUF
      if w[ns] is not None:
        w[ns].wait()
        w[ns] = None
      g[ns] = _gather(ni, bufs[ns], gsems[ns])
    g[s].wait()

    off = pbase + ci * CHUNK
    m0 = jnp.clip(total - off, 0, CHUNK)
    buf = bufs[s]

    def _zero_row(r, acc):
      for j in range(D // LANES):
        buf[r, pl.ds(j * LANES, LANES)] = zf
      return acc

    lax.fori_loop(m0, CHUNK, _zero_row, 0)

    w[s] = pltpu.async_copy(buf, out_hbm.at[pl.ds(b * L + off, CHUNK), :], wsems[s])
  for s in range(NBUF):
    if w[s] is not None:
      w[s].wait()


@jax.jit
def kernel(encoder_output, durations):
  enc_flat = encoder_output.reshape(B * T, D)
  dur32 = durations.astype(jnp.int32)
  mesh = plsc.VectorSubcoreMesh(
      core_axis_name="c", subcore_axis_name="s", num_cores=NC, num_subcores=NS
  )
  run = pl.kernel(
      _body,
      out_type=jax.ShapeDtypeStruct((B * L, D), jnp.float32),
      mesh=mesh,
      scratch_types=(
          [pltpu.VMEM((T,), jnp.int32), pltpu.VMEM((L,), jnp.int32)]
          + [pltpu.VMEM((CHUNK, D), jnp.float32)] * NBUF
          + [pltpu.SemaphoreType.DMA] * (2 * NBUF)
      ),
      compiler_params=pltpu.CompilerParams(needs_layout_passes=False),
  )
  out = run(enc_flat, dur32)
  return out.reshape(B, L, D)


# speedup vs baseline: 36.5795x; 2.0508x over previous
import jax
import jax.numpy as jnp
from jax import lax
from jax.experimental import pallas as pl
from jax.experimental.pallas import tpu as pltpu
from jax.experimental.pallas import tpu_sc as plsc

B, T, D = 16, 512, 256
L = 2048
MAX_DUR = 8

NC, NS = 2, 16
HALF = L // 2
CHUNK = 64
NCHUNK = HALF // CHUNK
SROWS = CHUNK + 8
LANES = 16


def _body(enc_hbm, dur_hbm, out_hbm, dur_v, idx_v, st0, st1, ob0, ob1,
          ssem0, ssem1, osem0, osem1):
  cid = lax.axis_index("c")
  sid = lax.axis_index("s")
  wid = sid * NC + cid
  b = wid // 2
  half = wid % 2
  pbase = half * HALF
  row0 = b * T

  pltpu.sync_copy(dur_hbm.at[b], dur_v)

  zi = jnp.full((LANES,), row0, jnp.int32)
  for j in range(HALF // LANES):
    idx_v[pl.ds(pbase + j * LANES, LANES)] = zi

  t_iota = lax.iota(jnp.int32, LANES)
  carry = jnp.int32(0)
  for i in range(T // LANES):
    v = dur_v[pl.ds(i * LANES, LANES)]
    inc = plsc.cumsum(v)
    start = inc - v + carry
    carry = carry + jnp.sum(v)
    val = t_iota + (row0 + i * LANES)
    for k in range(MAX_DUR - 1):
      pos = start + k
      mask = (v > k) & (pos < L)
      plsc.store_scatter(idx_v, [pos], val, mask=mask)
  total = carry

  zf = jnp.zeros((LANES,), jnp.float32)
  stbufs = (st0, st1)
  obufs = (ob0, ob1)
  ssems = (ssem0, ssem1)
  osems = (osem0, osem1)

  def _stage_base(ci):
    off = pbase + jnp.minimum(ci, NCHUNK - 1) * CHUNK
    t0g = idx_v[pl.ds(off, LANES)][0]
    sbl = jnp.clip(t0g - row0, 0, T - SROWS)
    return pl.multiple_of(jnp.bitwise_and(sbl, -8), 8)

  def _stage_issue(ci, sb, stbuf, sem):
    pltpu.async_copy(enc_hbm.at[pl.ds(row0 + sb, SROWS), :], stbuf, sem)

  sb_pair = []
  for par in range(2):
    sb = _stage_base(jnp.int32(par))
    _stage_issue(par, sb, stbufs[par], ssems[par])
    sb_pair.append(sb)

  def _outer(oi, sbs):
    new_sbs = []
    for par in range(2):
      ci = oi * 2 + par
      off = pbase + ci * CHUNK
      stb = stbufs[par]
      ob = obufs[par]
      sb = sbs[par]

      pltpu.make_async_copy(enc_hbm.at[pl.ds(row0, SROWS), :], stb, ssems[par]).wait()

      @pl.when(oi > 0)
      def _drain():
        pltpu.make_async_copy(ob, out_hbm.at[pl.ds(b * L + off, CHUNK), :], osems[par]).wait()

      base = row0 + sb
      for gg in range(CHUNK // LANES):
        lvec = idx_v[pl.ds(off + gg * LANES, LANES)] - base
        lvec = jnp.clip(lvec, 0, SROWS - 1)
        for l in range(LANES):
          lt = lvec[l]
          fr = gg * LANES + l
          for j in range(D // LANES):
            ob[fr, pl.ds(j * LANES, LANES)] = stb[lt, pl.ds(j * LANES, LANES)]

      sb_next = _stage_base(ci + 2)

      @pl.when(oi < NCHUNK // 2 - 1)
      def _refill():
        _stage_issue(ci + 2, sb_next, stb, ssems[par])

      new_sbs.append(sb_next)

      m0 = jnp.clip(total - off, 0, CHUNK)

      def _zero_row(r, acc):
        for j in range(D // LANES):
          ob[r, pl.ds(j * LANES, LANES)] = zf
        return acc

      lax.fori_loop(m0, CHUNK, _zero_row, 0)

      pltpu.async_copy(ob, out_hbm.at[pl.ds(b * L + off, CHUNK), :], osems[par])
    return tuple(new_sbs)

  lax.fori_loop(0, NCHUNK // 2, _outer, tuple(sb_pair))

  for par in range(2):
    pltpu.make_async_copy(obufs[par], out_hbm.at[pl.ds(b * L + pbase, CHUNK), :], osems[par]).wait()


@jax.jit
def kernel(encoder_output, durations):
  enc_flat = encoder_output.reshape(B * T, D)
  dur32 = durations.astype(jnp.int32)
  mesh = plsc.VectorSubcoreMesh(
      core_axis_name="c", subcore_axis_name="s", num_cores=NC, num_subcores=NS
  )
  run = pl.kernel(
      _body,
      out_type=jax.ShapeDtypeStruct((B * L, D), jnp.float32),
      mesh=mesh,
      scratch_types=(
          [pltpu.VMEM((T,), jnp.int32), pltpu.VMEM((L,), jnp.int32)]
          + [pltpu.VMEM((SROWS, D), jnp.float32)] * 2
          + [pltpu.VMEM((CHUNK, D), jnp.float32)] * 2
          + [pltpu.SemaphoreType.DMA] * 4
      ),
      compiler_params=pltpu.CompilerParams(needs_layout_passes=False),
  )
  out = run(enc_flat, dur32)
  return out.reshape(B, L, D)


# speedup vs baseline: 38.1225x; 1.0422x over previous
import jax
import jax.numpy as jnp
from jax import lax
from jax.experimental import pallas as pl
from jax.experimental.pallas import tpu as pltpu
from jax.experimental.pallas import tpu_sc as plsc

B, T, D = 16, 512, 256
L = 2048
MAX_DUR = 8

NC, NS = 2, 16
HALF = L // 2
CHUNK = 64
NCHUNK = HALF // CHUNK
SROWS = CHUNK + 8
ZROW = SROWS
LANES = 16


def _body(enc_hbm, dur_hbm, out_hbm, dur_v, idx_v, st0, st1, ob0, ob1,
          ssem0, ssem1, osem0, osem1):
  cid = lax.axis_index("c")
  sid = lax.axis_index("s")
  wid = sid * NC + cid
  b = wid // 2
  half = wid % 2
  row0 = b * T

  pltpu.sync_copy(dur_hbm.at[b], dur_v)

  zf = jnp.zeros((LANES,), jnp.float32)
  stbufs = (st0, st1)
  obufs = (ob0, ob1)
  ssems = (ssem0, ssem1)
  osems = (osem0, osem1)

  for stb in stbufs:
    for j in range(D // LANES):
      stb[ZROW, pl.ds(j * LANES, LANES)] = zf

  zi = jnp.full((LANES,), row0, jnp.int32)
  for j in range(L // LANES):
    idx_v[pl.ds(j * LANES, LANES)] = zi

  t_iota = lax.iota(jnp.int32, LANES)
  carry = jnp.int32(0)
  for i in range(T // LANES):
    v = dur_v[pl.ds(i * LANES, LANES)]
    inc = plsc.cumsum(v)
    start = inc - v + carry
    carry = carry + inc[LANES - 1]
    val = t_iota + (row0 + i * LANES)
    for k in range(MAX_DUR - 1):
      pos = start + k
      mask = (v > k) & (pos < L)
      plsc.store_scatter(idx_v, [pos], val, mask=mask)
  total = carry

  def _off(ci):
    return (2 * ci + half) * CHUNK

  def _stage_base(ci):
    off = _off(jnp.minimum(ci, NCHUNK - 1))
    t0g = idx_v[pl.ds(off, LANES)][0]
    sbl = jnp.clip(t0g - row0, 0, T - SROWS)
    return pl.multiple_of(jnp.bitwise_and(sbl, -8), 8)

  def _stage_issue(sb, stbuf, sem):
    pltpu.async_copy(
        enc_hbm.at[pl.ds(row0 + sb, SROWS), :], stbuf.at[pl.ds(0, SROWS), :], sem
    )

  sb_pair = []
  for par in range(2):
    sb = _stage_base(jnp.int32(par))
    _stage_issue(sb, stbufs[par], ssems[par])
    sb_pair.append(sb)

  def _outer(oi, sbs):
    new_sbs = []
    for par in range(2):
      ci = oi * 2 + par
      off = _off(ci)
      stb = stbufs[par]
      ob = obufs[par]
      sb = sbs[par]

      pltpu.make_async_copy(
          enc_hbm.at[pl.ds(row0, SROWS), :], stb.at[pl.ds(0, SROWS), :], ssems[par]
      ).wait()

      @pl.when(oi > 0)
      def _drain():
        pltpu.make_async_copy(ob, out_hbm.at[pl.ds(b * L + off, CHUNK), :], osems[par]).wait()

      base = row0 + sb
      for gg in range(CHUNK // LANES):
        lvec = idx_v[pl.ds(off + gg * LANES, LANES)] - base
        lvec = jnp.clip(lvec, 0, SROWS - 1)
        masked = (off + gg * LANES) + t_iota >= total
        lvec = jnp.where(masked, ZROW, lvec)
        for l in range(LANES):
          lt = lvec[l]
          fr = gg * LANES + l
          for j in range(D // LANES):
            ob[fr, pl.ds(j * LANES, LANES)] = stb[lt, pl.ds(j * LANES, LANES)]

      sb_next = _stage_base(ci + 2)

      @pl.when(oi < NCHUNK // 2 - 1)
      def _refill():
        _stage_issue(sb_next, stb, ssems[par])

      new_sbs.append(sb_next)

      pltpu.async_copy(ob, out_hbm.at[pl.ds(b * L + off, CHUNK), :], osems[par])
    return tuple(new_sbs)

  lax.fori_loop(0, NCHUNK // 2, _outer, tuple(sb_pair))

  for par in range(2):
    pltpu.make_async_copy(obufs[par], out_hbm.at[pl.ds(b * L, CHUNK), :], osems[par]).wait()


@jax.jit
def kernel(encoder_output, durations):
  enc_flat = encoder_output.reshape(B * T, D)
  dur32 = durations.astype(jnp.int32)
  mesh = plsc.VectorSubcoreMesh(
      core_axis_name="c", subcore_axis_name="s", num_cores=NC, num_subcores=NS
  )
  run = pl.kernel(
      _body,
      out_type=jax.ShapeDtypeStruct((B * L, D), jnp.float32),
      mesh=mesh,
      scratch_types=(
          [pltpu.VMEM((T,), jnp.int32), pltpu.VMEM((L,), jnp.int32)]
          + [pltpu.VMEM((SROWS + 1, D), jnp.float32)] * 2
          + [pltpu.VMEM((CHUNK, D), jnp.float32)] * 2
          + [pltpu.SemaphoreType.DMA] * 4
      ),
      compiler_params=pltpu.CompilerParams(needs_layout_passes=False),
  )
  out = run(enc_flat, dur32)
  return out.reshape(B, L, D)


# speedup vs baseline: 45.2743x; 1.1876x over previous
import jax
import jax.numpy as jnp
from jax import lax
from jax.experimental import pallas as pl
from jax.experimental.pallas import tpu as pltpu
from jax.experimental.pallas import tpu_sc as plsc

B, T, D = 16, 512, 256
L = 2048
MAX_DUR = 8

NC, NS = 2, 16
HALF = L // 2
CHUNK = 64
NCHUNK = HALF // CHUNK
SROWS = CHUNK + 8
ZROW = SROWS
LANES = 16


def _body(enc_hbm, dur_hbm, out_hbm, dur_v, idx_v, st0, st1, ob0, ob1,
          ssem0, ssem1, osem0, osem1):
  cid = lax.axis_index("c")
  sid = lax.axis_index("s")
  wid = sid * NC + cid
  b = wid // 2
  half = wid % 2
  row0 = b * T

  pltpu.sync_copy(dur_hbm.at[b], dur_v)

  zf = jnp.zeros((LANES,), jnp.float32)
  stbufs = (st0, st1)
  obufs = (ob0, ob1)
  ssems = (ssem0, ssem1)
  osems = (osem0, osem1)

  for stb in stbufs:
    for j in range(D // LANES):
      stb[ZROW, pl.ds(j * LANES, LANES)] = zf

  zi = jnp.full((LANES,), row0, jnp.int32)
  for j in range(L // LANES):
    idx_v[pl.ds(j * LANES, LANES)] = zi

  t_iota = lax.iota(jnp.int32, LANES)
  carry = jnp.int32(0)
  for i in range(T // LANES):
    v = dur_v[pl.ds(i * LANES, LANES)]
    inc = plsc.cumsum(v)
    start = inc - v + carry
    carry = carry + inc[LANES - 1]
    val = t_iota + (row0 + i * LANES)
    for k in range(MAX_DUR - 1):
      pos = start + k
      mask = (v > k) & (pos < L)
      plsc.store_scatter(idx_v, [pos], val, mask=mask)
  total = carry

  def _off(ci):
    return (2 * ci + half) * CHUNK

  def _stage_base(ci):
    off = _off(jnp.minimum(ci, NCHUNK - 1))
    t0g = idx_v[pl.ds(off, LANES)][0]
    sbl = jnp.clip(t0g - row0, 0, T - SROWS)
    return pl.multiple_of(jnp.bitwise_and(sbl, -8), 8)

  def _stage_issue(sb, stbuf, sem):
    pltpu.async_copy(
        enc_hbm.at[pl.ds(row0 + sb, SROWS), :], stbuf.at[pl.ds(0, SROWS), :], sem
    )

  sb_pair = []
  for par in range(2):
    sb = _stage_base(jnp.int32(par))
    _stage_issue(sb, stbufs[par], ssems[par])
    sb_pair.append(sb)

  def _outer(oi, sbs):
    new_sbs = []
    for par in range(2):
      ci = oi * 2 + par
      off = _off(ci)
      stb = stbufs[par]
      ob = obufs[par]
      sb = sbs[par]

      pltpu.make_async_copy(
          enc_hbm.at[pl.ds(row0, SROWS), :], stb.at[pl.ds(0, SROWS), :], ssems[par]
      ).wait()

      @pl.when(oi > 0)
      def _drain():
        pltpu.make_async_copy(ob, out_hbm.at[pl.ds(b * L + off, CHUNK), :], osems[par]).wait()

      base = row0 + sb
      for gg in range(CHUNK // LANES):
        lvec = idx_v[pl.ds(off + gg * LANES, LANES)] - base
        lvec = jnp.clip(lvec, 0, SROWS - 1)
        masked = (off + gg * LANES) + t_iota >= total
        lvec = jnp.where(masked, ZROW, lvec)
        for l in range(LANES):
          lt = lvec[l]
          fr = gg * LANES + l
          vals = [stb[lt, pl.ds(j * LANES, LANES)] for j in range(D // LANES)]
          for j in range(D // LANES):
            ob[fr, pl.ds(j * LANES, LANES)] = vals[j]

      sb_next = _stage_base(ci + 2)

      @pl.when(oi < NCHUNK // 2 - 1)
      def _refill():
        _stage_issue(sb_next, stb, ssems[par])

      new_sbs.append(sb_next)

      pltpu.async_copy(ob, out_hbm.at[pl.ds(b * L + off, CHUNK), :], osems[par])
    return tuple(new_sbs)

  lax.fori_loop(0, NCHUNK // 2, _outer, tuple(sb_pair))

  for par in range(2):
    pltpu.make_async_copy(obufs[par], out_hbm.at[pl.ds(b * L, CHUNK), :], osems[par]).wait()


@jax.jit
def kernel(encoder_output, durations):
  enc_flat = encoder_output.reshape(B * T, D)
  dur32 = durations.astype(jnp.int32)
  mesh = plsc.VectorSubcoreMesh(
      core_axis_name="c", subcore_axis_name="s", num_cores=NC, num_subcores=NS
  )
  run = pl.kernel(
      _body,
      out_type=jax.ShapeDtypeStruct((B * L, D), jnp.float32),
      mesh=mesh,
      scratch_types=(
          [pltpu.VMEM((T,), jnp.int32), pltpu.VMEM((L,), jnp.int32)]
          + [pltpu.VMEM((SROWS + 1, D), jnp.float32)] * 2
          + [pltpu.VMEM((CHUNK, D), jnp.float32)] * 2
          + [pltpu.SemaphoreType.DMA] * 4
      ),
      compiler_params=pltpu.CompilerParams(needs_layout_passes=False),
  )
  out = run(enc_flat, dur32)
  return out.reshape(B, L, D)
